# baseline (speedup 1.0000x reference)
import jax
import jax.numpy as jnp
from jax import lax
from jax.experimental import pallas as pl
from jax.experimental.pallas import tpu as pltpu
from jax.experimental.pallas import tpu_sc as plsc

WEEK = 7
DAY = 288
D = 64
LANES = 16
WINDOW = 128
UNROLL = 8


def _sc_lookup(dow_table, tod_table, dow_flat, tod_flat, n_rows):
    mesh = plsc.VectorSubcoreMesh(core_axis_name="c", subcore_axis_name="s")

    @pl.kernel(
        out_type=jax.ShapeDtypeStruct((n_rows, D), jnp.float32),
        mesh=mesh,
        scratch_types=[
            pltpu.VMEM((WEEK, D), jnp.float32),
            pltpu.VMEM((DAY, D), jnp.float32),
        ],
        compiler_params=pltpu.CompilerParams(
            use_tc_tiling_on_sc=False, needs_layout_passes=False
        ),
    )
    def k(dowt_hbm, todt_hbm, dow_hbm, tod_hbm, out_hbm, dowt_ref, todt_ref):
        pltpu.sync_copy(dowt_hbm, dowt_ref)
        pltpu.sync_copy(todt_hbm, todt_ref)

        def body(dow_v, tod_v, out_v):
            @pl.loop(0, WINDOW, step=LANES)
            def _(r0):
                dvec = dow_v.at[0][pl.ds(r0, LANES)]
                tvec = tod_v.at[0][pl.ds(r0, LANES)]
                rows = lax.iota(jnp.int32, LANES) + r0

                @plsc.parallel_loop(0, D, 1, unroll=UNROLL)
                def _(d):
                    col = jnp.full((LANES,), d, jnp.int32)
                    val = plsc.load_gather(dowt_ref, [dvec, col]) + plsc.load_gather(
                        todt_ref, [tvec, col]
                    )
                    plsc.store_scatter(out_v, [rows, col], val)

        pltpu.emit_pipeline(
            body,
            grid=(n_rows // WINDOW,),
            in_specs=[
                pl.BlockSpec((1, WINDOW), index_map=lambda i: (0, i)),
                pl.BlockSpec((1, WINDOW), index_map=lambda i: (0, i)),
            ],
            out_specs=[pl.BlockSpec((WINDOW, D), index_map=lambda i: (i, 0))],
            core_axis_name=("c", "s"),
            dimension_semantics=(pltpu.PARALLEL,),
        )(dow_hbm, tod_hbm, out_hbm)

    return k(dow_table, tod_table, dow_flat, tod_flat)


@jax.jit
def kernel(dow, tod, dow_table, tod_table):
    b, t = dow.shape
    n = b * t
    dow_flat = dow.reshape(1, n).astype(jnp.int32)
    tod_flat = tod.reshape(1, n).astype(jnp.int32)
    out = _sc_lookup(dow_table, tod_table, dow_flat, tod_flat, n)
    return out.reshape(b, t, D)


# speedup vs baseline: 2.8904x; 2.8904x over previous
import jax
import jax.numpy as jnp
from jax import lax
from jax.experimental import pallas as pl
from jax.experimental.pallas import tpu as pltpu
from jax.experimental.pallas import tpu_sc as plsc

WEEK = 7
DAY = 288
D = 64
LANES = 16
WINDOW = 128


def _build_combined_table(dow_table, tod_table):
    def body(dow_ref, tod_ref, c_ref):
        c_ref[...] = dow_ref[...][:, None, :] + tod_ref[...][None, :, :]

    return pl.pallas_call(
        body,
        out_shape=jax.ShapeDtypeStruct((WEEK, DAY, D), jnp.float32),
    )(dow_table, tod_table)


def _sc_gather(combined, dow_flat, tod_flat, n_rows):
    mesh = plsc.VectorSubcoreMesh(core_axis_name="c", subcore_axis_name="s")

    @pl.kernel(
        out_type=jax.ShapeDtypeStruct((n_rows, D), jnp.float32),
        mesh=mesh,
        scratch_types=[
            pltpu.VMEM((WINDOW,), jnp.int32),
            pltpu.VMEM_SHARED((WEEK * DAY, D), jnp.float32),
        ],
        compiler_params=pltpu.CompilerParams(use_tc_tiling_on_sc=False),
    )
    def k(c_hbm, dow_hbm, tod_hbm, out_hbm, idx_ref, c_shared):
        @pl.when(lax.axis_index("s") == 0)
        def _():
            pltpu.sync_copy(c_hbm, c_shared)

        plsc.subcore_barrier()

        def body(dow_v, tod_v, out_v):
            @pl.loop(0, WINDOW, step=LANES)
            def _(i):
                sl = pl.ds(i, LANES)
                idx_ref[sl] = dow_v.at[0][sl] * DAY + tod_v.at[0][sl]

            pltpu.sync_copy(c_shared.at[idx_ref], out_v)

        pltpu.emit_pipeline(
            body,
            grid=(n_rows // WINDOW,),
            in_specs=[
                pl.BlockSpec((1, WINDOW), index_map=lambda i: (0, i)),
                pl.BlockSpec((1, WINDOW), index_map=lambda i: (0, i)),
            ],
            out_specs=[pl.BlockSpec((WINDOW, D), index_map=lambda i: (i, 0))],
            core_axis_name=("c", "s"),
            dimension_semantics=(pltpu.PARALLEL,),
        )(dow_hbm, tod_hbm, out_hbm)

    return k(combined, dow_flat, tod_flat)


@jax.jit
def kernel(dow, tod, dow_table, tod_table):
    b, t = dow.shape
    n = b * t
    combined = _build_combined_table(dow_table, tod_table).reshape(WEEK * DAY, D)
    dow_flat = dow.reshape(1, n).astype(jnp.int32)
    tod_flat = tod.reshape(1, n).astype(jnp.int32)
    out = _sc_gather(combined, dow_flat, tod_flat, n)
    return out.reshape(b, t, D)


# speedup vs baseline: 3.1421x; 1.0871x over previous
import jax
import jax.numpy as jnp
from jax import lax
from jax.experimental import pallas as pl
from jax.experimental.pallas import tpu as pltpu
from jax.experimental.pallas import tpu_sc as plsc

WEEK = 7
DAY = 288
D = 64
LANES = 16
WINDOW = 100


def _build_combined_table(dow_table, tod_table):
    def body(dow_ref, tod_ref, c_ref):
        c_ref[...] = dow_ref[...][:, None, :] + tod_ref[...][None, :, :]

    return pl.pallas_call(
        body,
        out_shape=jax.ShapeDtypeStruct((WEEK, DAY, D), jnp.float32),
    )(dow_table, tod_table)


def _sc_gather(combined, dow3, tod3, b, t):
    mesh = plsc.VectorSubcoreMesh(core_axis_name="c", subcore_axis_name="s")
    n_chunks = t // WINDOW

    @pl.kernel(
        out_type=jax.ShapeDtypeStruct((b, t, D), jnp.float32),
        mesh=mesh,
        scratch_types=[
            pltpu.VMEM((WINDOW,), jnp.int32),
            pltpu.VMEM_SHARED((WEEK * DAY, D), jnp.float32),
        ],
        compiler_params=pltpu.CompilerParams(use_tc_tiling_on_sc=False),
    )
    def k(c_hbm, dow_hbm, tod_hbm, out_hbm, idx_ref, c_shared):
        @pl.when(lax.axis_index("s") == 0)
        def _():
            pltpu.sync_copy(c_hbm, c_shared)

        plsc.subcore_barrier()

        def body(dow_v, tod_v, out_v):
            @pl.loop(0, WINDOW - LANES + 1, step=LANES)
            def _(i):
                sl = pl.ds(i, LANES)
                idx_ref[sl] = dow_v.at[0, 0][sl] * DAY + tod_v.at[0, 0][sl]

            tail = pl.ds(WINDOW - LANES, LANES)
            idx_ref[tail] = dow_v.at[0, 0][tail] * DAY + tod_v.at[0, 0][tail]

            pltpu.sync_copy(c_shared.at[idx_ref], out_v.at[0])

        pltpu.emit_pipeline(
            body,
            grid=(b, n_chunks),
            in_specs=[
                pl.BlockSpec((1, 1, WINDOW), index_map=lambda i, j: (i, j, 0)),
                pl.BlockSpec((1, 1, WINDOW), index_map=lambda i, j: (i, j, 0)),
            ],
            out_specs=[
                pl.BlockSpec((1, WINDOW, D), index_map=lambda i, j: (i, j, 0))
            ],
            core_axis_name=("c", "s"),
            dimension_semantics=(pltpu.PARALLEL, pltpu.PARALLEL),
        )(dow_hbm, tod_hbm, out_hbm)

    return k(combined, dow3, tod3)


@jax.jit
def kernel(dow, tod, dow_table, tod_table):
    b, t = dow.shape
    combined = _build_combined_table(dow_table, tod_table).reshape(WEEK * DAY, D)
    dow3 = dow.reshape(b, t // WINDOW, WINDOW).astype(jnp.int32)
    tod3 = tod.reshape(b, t // WINDOW, WINDOW).astype(jnp.int32)
    return _sc_gather(combined, dow3, tod3, b, t)
